# baseline (speedup 1.0000x reference)
import functools

import jax
import jax.numpy as jnp
from jax import lax
from jax.experimental import pallas as pl
from jax.experimental.pallas import tpu as pltpu
from jax.experimental.pallas import tpu_sc as plsc

_NUM_CORES = 2
_NUM_SUBCORES = 16
_NUM_WORKERS = _NUM_CORES * _NUM_SUBCORES
_LANES = 16
_TEMPLATE_ROWS = 64


@functools.lru_cache(maxsize=None)
def _build(batch: int, d: int):
    assert batch % _NUM_WORKERS == 0
    b_per_w = batch // _NUM_WORKERS
    w = min(_TEMPLATE_ROWS, b_per_w)
    assert b_per_w % w == 0
    k_chunks = b_per_w // w

    mesh = plsc.VectorSubcoreMesh(
        core_axis_name="c",
        subcore_axis_name="s",
        num_cores=_NUM_CORES,
        num_subcores=_NUM_SUBCORES,
    )

    @functools.partial(
        pl.kernel,
        out_type=jax.ShapeDtypeStruct((batch * d,), jnp.float32),
        mesh=mesh,
        scratch_types=[
            pltpu.VMEM((w * d,), jnp.float32),
            pltpu.SemaphoreType.DMA,
        ],
    )
    def tiled_broadcast(emb_hbm, out_hbm, buf, sem):
        wid = lax.axis_index("s") * _NUM_CORES + lax.axis_index("c")
        base = wid * (b_per_w * d)
        pltpu.sync_copy(emb_hbm, buf.at[pl.ds(0, d)])
        row = [buf[pl.ds(c * _LANES, _LANES)] for c in range(d // _LANES)]
        for r in range(1, w):
            for c in range(d // _LANES):
                buf[pl.ds(r * d + c * _LANES, _LANES)] = row[c]
        copies = [
            pltpu.async_copy(
                buf, out_hbm.at[pl.ds(base + k * (w * d), w * d)], sem
            )
            for k in range(k_chunks)
        ]
        for cp in copies:
            cp.wait()

    return tiled_broadcast


def kernel(inputs, embedding):
    batch = inputs.shape[0]
    d = embedding.shape[1]
    flat = _build(batch, d)(embedding.reshape(d))
    return flat.reshape(batch, d)


# speedup vs baseline: 4.9295x; 4.9295x over previous
import functools

import jax
import jax.numpy as jnp
from jax.experimental import pallas as pl
from jax.experimental.pallas import tpu as pltpu


@functools.lru_cache(maxsize=None)
def _build_tc(batch: int, d: int, block: int):
    grid = batch // block

    def body(emb_ref, out_ref):
        out_ref[...] = jnp.broadcast_to(emb_ref[...], (block, d))

    return pl.pallas_call(
        body,
        grid=(grid,),
        in_specs=[pl.BlockSpec((1, d), lambda i: (0, 0))],
        out_specs=pl.BlockSpec((block, d), lambda i: (i, 0)),
        out_shape=jax.ShapeDtypeStruct((batch, d), jnp.float32),
    )


def kernel(inputs, embedding):
    batch = inputs.shape[0]
    d = embedding.shape[1]
    return _build_tc(batch, d, 2048)(embedding)


# speedup vs baseline: 6.6019x; 1.3393x over previous
import functools

import jax
import jax.numpy as jnp
from jax.experimental import pallas as pl
from jax.experimental.pallas import tpu as pltpu


@functools.lru_cache(maxsize=None)
def _build_tc(batch: int, d: int, w: int):
    k_chunks = batch // w

    def body(emb_ref, out_hbm, scratch, sem):
        scratch[...] = jnp.broadcast_to(emb_ref[...], (w, d))
        copies = [
            pltpu.make_async_copy(scratch, out_hbm.at[pl.ds(k * w, w)], sem)
            for k in range(k_chunks)
        ]
        for cp in copies:
            cp.start()
        for cp in copies:
            cp.wait()

    return pl.pallas_call(
        body,
        in_specs=[pl.BlockSpec(memory_space=pltpu.VMEM)],
        out_specs=pl.BlockSpec(memory_space=pl.ANY),
        out_shape=jax.ShapeDtypeStruct((batch, d), jnp.float32),
        scratch_shapes=[
            pltpu.VMEM((w, d), jnp.float32),
            pltpu.SemaphoreType.DMA,
        ],
    )


def kernel(inputs, embedding):
    batch = inputs.shape[0]
    d = embedding.shape[1]
    return _build_tc(batch, d, 2048)(embedding)


# speedup vs baseline: 6.6951x; 1.0141x over previous
import functools

import jax
import jax.numpy as jnp
from jax.experimental import pallas as pl
from jax.experimental.pallas import tpu as pltpu


@functools.lru_cache(maxsize=None)
def _build_tc(batch: int, d: int, w: int):
    k_chunks = batch // w

    def body(emb_ref, out_hbm, scratch, sem):
        scratch[...] = jnp.broadcast_to(emb_ref[...], (w, d))
        copies = [
            pltpu.make_async_copy(scratch, out_hbm.at[pl.ds(k * w, w)], sem)
            for k in range(k_chunks)
        ]
        for cp in copies:
            cp.start()
        for cp in copies:
            cp.wait()

    return pl.pallas_call(
        body,
        in_specs=[pl.BlockSpec(memory_space=pltpu.VMEM)],
        out_specs=pl.BlockSpec(memory_space=pl.ANY),
        out_shape=jax.ShapeDtypeStruct((batch, d), jnp.float32),
        scratch_shapes=[
            pltpu.VMEM((w, d), jnp.float32),
            pltpu.SemaphoreType.DMA,
        ],
    )


def kernel(inputs, embedding):
    batch = inputs.shape[0]
    d = embedding.shape[1]
    return _build_tc(batch, d, 512)(embedding)
